# Initial kernel scaffold; baseline (speedup 1.0000x reference)
import jax
import jax.numpy as jnp
from jax.experimental import pallas as pl


def kernel(last_coors, last_features, current_coors, edge, params_in, params_out):
    raise NotImplementedError("write your pallas kernel here")


# speedup vs baseline: 1.2038x; 1.2038x over previous
import functools

import jax
import jax.numpy as jnp
from jax import lax
from jax.experimental import pallas as pl
from jax.experimental.pallas import tpu as pltpu
from jax.experimental.pallas import tpu_sc as plsc

N_NODES = 10000
N_EDGES = 160000
NC, NS, L = 2, 16, 16
NW = NC * NS
EPS = 1e-5

FW = 8
EPTP = 5120
NE_PAD = NW * EPTP
GCH = 1280
NGCH = EPTP // GCH

NROW = 320
NPAD = NW * NROW
ACCW = 304
Y4W = 384
CH = 1280
NCHUNK = N_EDGES // CH
NVREG = CH // L
G = 32
MB = 192


def _gather_body(tab_a, tab_b, src_ids, dst_ids, out, ta, tb, idx_s, idx_d,
                 stage, sem):
    w = lax.axis_index("c") * NS + lax.axis_index("s")
    pltpu.sync_copy(tab_a, ta)
    pltpu.sync_copy(tab_b, tb)
    iota = lax.iota(jnp.int32, L)
    zero = jnp.zeros((L,), jnp.float32)

    for c in range(NGCH):
        base = w * EPTP + c * GCH
        pltpu.sync_copy(src_ids.at[pl.ds(base, GCH)], idx_s)
        pltpu.sync_copy(dst_ids.at[pl.ds(base, GCH)], idx_d)

        def vbody(i, _):
            e16 = i * L + iota
            isrc = idx_s[pl.ds(i * L, L)]
            idst = idx_d[pl.ds(i * L, L)]
            for j in range(4):
                va = plsc.load_gather(ta, [j * N_NODES + isrc])
                plsc.store_scatter(stage, [jnp.full((L,), j, jnp.int32), e16],
                                   va)
            for j in range(3):
                va = plsc.load_gather(ta, [(4 + j) * N_NODES + isrc])
                vb = plsc.load_gather(tb, [j * N_NODES + idst])
                plsc.store_scatter(stage,
                                   [jnp.full((L,), 4 + j, jnp.int32), e16],
                                   va - vb)
            plsc.store_scatter(stage, [jnp.full((L,), 7, jnp.int32), e16],
                               zero)
            return 0

        lax.fori_loop(0, GCH // L, vbody, 0)
        pltpu.sync_copy(stage, out.at[:, pl.ds(base, GCH)])


def _sc_gather(tab_a, tab_b, src_ids, dst_ids):
    mesh = plsc.VectorSubcoreMesh(core_axis_name="c", subcore_axis_name="s",
                                  num_cores=NC, num_subcores=NS)
    return pl.kernel(
        _gather_body,
        out_type=jax.ShapeDtypeStruct((FW, NE_PAD), jnp.float32),
        mesh=mesh,
        compiler_params=pltpu.CompilerParams(needs_layout_passes=False),
        scratch_types=[
            pltpu.VMEM((7 * N_NODES,), jnp.float32),
            pltpu.VMEM((3 * N_NODES,), jnp.float32),
            pltpu.VMEM((GCH,), jnp.int32),
            pltpu.VMEM((GCH,), jnp.int32),
            pltpu.VMEM((FW, GCH), jnp.float32),
            pltpu.SemaphoreType.DMA,
        ],
    )(tab_a, tab_b, src_ids, dst_ids)


def _scatter_body(dst_hbm, y4_hbm, out, acc, ids_v, midx, mnl, tmp, rows_v,
                  sem):
    w = lax.axis_index("c") * NS + lax.axis_index("s")
    lo = w * NROW
    iota = lax.iota(jnp.int32, L)
    neg1 = jnp.full((L,), -1.0, jnp.float32)
    dummy = jnp.full((L,), NROW, jnp.int32)
    zero_i = jnp.zeros((L,), jnp.int32)

    def init_body(i, _):
        acc[pl.ds(i * L, L)] = neg1
        return 0

    lax.fori_loop(0, (NROW + 1) * ACCW // L, init_body, 0)
    for q in range(MB // G):
        for r in range(G // L):
            midx[q, pl.ds(r * L, L)] = zero_i

    def chunk_body(c, _):
        pltpu.sync_copy(dst_hbm.at[pl.ds(c * CH, CH)], ids_v)
        for q in range(MB // L):
            mnl[pl.ds(q * L, L)] = dummy

        def scan_body(i, cnt_vec):
            v = ids_v[pl.ds(i * L, L)]
            m = (v >= lo) & (v < lo + NROW)
            cs = plsc.cumsum(m.astype(jnp.int32))
            pos = cnt_vec + cs - 1
            eidx = c * CH + i * L + iota
            plsc.store_scatter(midx, [pos >> 5, pos & (G - 1)], eidx,
                               mask=m)
            plsc.store_scatter(mnl, [pos], v - lo, mask=m)
            tmp[...] = cs
            last = plsc.load_gather(tmp, [jnp.full((L,), L - 1, jnp.int32)])
            return cnt_vec + last

        cnt_vec = lax.fori_loop(0, NVREG, scan_body, jnp.zeros((L,), jnp.int32))
        cnt = jnp.max(cnt_vec)
        ngroups = (cnt + (G - 1)) // G

        def flush_body(g, _):
            cp = pltpu.async_copy(y4_hbm.at[midx.at[g]], rows_v, sem)
            cp.wait()

            def rmw_body(pp, _):
                sg = pp >> 4
                p = pp & (L - 1)
                rot = (iota + p) & (L - 1)
                nlr = plsc.load_gather(mnl, [g * G + sg * L + rot])
                rr = sg * L + rot
                abase = nlr * ACCW + iota
                for j in range(ACCW // L):
                    cols = j * L + iota
                    a = plsc.load_gather(acc, [abase + j * L])
                    b = plsc.load_gather(rows_v, [rr, cols])
                    plsc.store_scatter(acc, [abase + j * L],
                                       jnp.maximum(a, b))
                return 0

            lax.fori_loop(0, G, rmw_body, 0)
            return 0

        lax.fori_loop(0, ngroups, flush_body, 0)
        return 0

    lax.fori_loop(0, NCHUNK, chunk_body, 0)
    pltpu.sync_copy(acc.at[pl.ds(0, NROW * ACCW)],
                    out.at[pl.ds(lo * ACCW, NROW * ACCW)])


def _sc_scatter_max(dst_ids, y4):
    mesh = plsc.VectorSubcoreMesh(core_axis_name="c", subcore_axis_name="s",
                                  num_cores=NC, num_subcores=NS)
    return pl.kernel(
        _scatter_body,
        out_type=jax.ShapeDtypeStruct((NPAD * ACCW,), jnp.float32),
        mesh=mesh,
        compiler_params=pltpu.CompilerParams(needs_layout_passes=False),
        scratch_types=[
            pltpu.VMEM(((NROW + 1) * ACCW,), jnp.float32),
            pltpu.VMEM((CH,), jnp.int32),
            pltpu.VMEM((MB // G, G), jnp.int32),
            pltpu.VMEM((MB,), jnp.int32),
            pltpu.VMEM((L,), jnp.int32),
            pltpu.VMEM((G, Y4W), jnp.float32),
            pltpu.SemaphoreType.DMA,
        ],
    )(dst_ids, y4)


BLK = 6400
NBLK = N_EDGES // BLK


def _pass_body(nlayers, write_y, *refs):
    k = nlayers
    x_ref = refs[0]
    w_refs = refs[1:1 + k]
    b_refs = refs[1 + k:1 + 2 * k]
    s_refs = refs[1 + 2 * k:3 * k]
    t_refs = refs[3 * k:4 * k - 1]
    if write_y:
        y_ref = refs[4 * k - 1]
        stats_ref = refs[4 * k]
    else:
        stats_ref = refs[4 * k - 1]
    i = pl.program_id(0)
    xt = x_ref[...]
    a = lax.dot_general(xt, w_refs[0][...], (((0,), (0,)), ((), ())),
                        preferred_element_type=jnp.float32)
    a = jnp.maximum(a + b_refs[0][...], 0.0)
    for li in range(1, k):
        h = a * s_refs[li - 1][...] + t_refs[li - 1][...]
        a = jnp.dot(h, w_refs[li][...], preferred_element_type=jnp.float32)
        a = jnp.maximum(a + b_refs[li][...], 0.0)
    dout = a.shape[1]
    ssum = jnp.sum(a, axis=0, keepdims=True)
    ssq = jnp.sum(a * a, axis=0, keepdims=True)
    st = jnp.concatenate(
        [ssum, ssq, jnp.zeros((6, dout), jnp.float32)], axis=0)

    @pl.when(i == 0)
    def _():
        stats_ref[...] = jnp.zeros_like(stats_ref)

    stats_ref[...] += st
    if write_y:
        y_ref[...] = a


def _mlp_pass(x0t, ws, bs, ss, ts, write_y):
    k = len(ws)
    dout = ws[-1].shape[1]
    full = lambda shape: pl.BlockSpec(shape, lambda i: (0, 0))
    in_specs = [pl.BlockSpec((FW, BLK), lambda i: (0, i))]
    in_specs += [full(w.shape) for w in ws]
    in_specs += [full((1, b.shape[1])) for b in bs]
    in_specs += [full((1, s.shape[1])) for s in ss]
    in_specs += [full((1, t.shape[1])) for t in ts]
    out_shapes = []
    out_specs = []
    if write_y:
        out_shapes.append(jax.ShapeDtypeStruct((N_EDGES, dout), jnp.float32))
        out_specs.append(pl.BlockSpec((BLK, dout), lambda i: (i, 0)))
    out_shapes.append(jax.ShapeDtypeStruct((8, dout), jnp.float32))
    out_specs.append(full((8, dout)))
    out = pl.pallas_call(
        functools.partial(_pass_body, k, write_y),
        grid=(NBLK,),
        in_specs=in_specs,
        out_specs=out_specs,
        out_shape=out_shapes,
    )(x0t, *ws, *bs, *ss, *ts)
    return tuple(out) if write_y else out[0]


def _out_body(x_ref, s4_ref, t4_ref, w_ref, b_ref, g_ref, be_ref, o_ref):
    x = x_ref[pl.ds(0, N_NODES), :]
    xb = jnp.where(x < 0.0, 0.0, x * s4_ref[...] + t4_ref[...])
    z = jnp.dot(xb, w_ref[...], preferred_element_type=jnp.float32)
    z = jnp.maximum(z + b_ref[...], 0.0)
    n = z.shape[0]
    mean = jnp.sum(z, axis=0, keepdims=True) / n
    var = jnp.sum(z * z, axis=0, keepdims=True) / n - mean * mean
    inv = g_ref[...] * lax.rsqrt(var + EPS)
    o_ref[...] = (z - mean) * inv + be_ref[...]


def _out_mlp(xagg, s4, t4, w5, b5, g5, be5):
    return pl.pallas_call(
        _out_body,
        out_shape=jax.ShapeDtypeStruct((N_NODES, ACCW), jnp.float32),
    )(xagg, s4, t4, w5, b5, g5, be5)


def _fold(stats, gamma, beta, n):
    mean = stats[0] / n
    var = stats[1] / n - mean * mean
    s = gamma * lax.rsqrt(var + EPS)
    t = beta - mean * s
    return s[None, :], t[None, :]


def kernel(last_coors, last_features, current_coors, edge, params_in,
           params_out):
    f32 = jnp.float32
    tab_a = jnp.concatenate([last_features.T, last_coors.T],
                            axis=0).reshape(-1)
    tab_b = current_coors.T.reshape(-1)
    dst_ids = edge[0]
    pad = jnp.zeros((NE_PAD - N_EDGES,), jnp.int32)
    src_pad = jnp.concatenate([edge[1], pad])
    dst_pad = jnp.concatenate([dst_ids, pad])

    featsT = _sc_gather(tab_a, tab_b, src_pad, dst_pad)

    ws, bs, gs, bes = [], [], [], []
    for (wt, b, g, be) in params_in:
        ws.append(wt.T)
        bs.append(b[None, :])
        gs.append(g)
        bes.append(be)
    w1 = jnp.zeros((FW, ws[0].shape[1]), f32).at[0:7, :].set(ws[0])
    w4 = jnp.zeros((ws[3].shape[0], Y4W), f32).at[:, 0:300].set(ws[3])
    b4 = jnp.zeros((1, Y4W), f32).at[:, 0:300].set(bs[3])
    g4 = jnp.zeros((Y4W,), f32).at[0:300].set(gs[3])
    be4 = jnp.zeros((Y4W,), f32).at[0:300].set(bes[3])
    ws = [w1, ws[1], ws[2], w4]
    bs = [bs[0], bs[1], bs[2], b4]

    n = float(N_EDGES)
    st1 = _mlp_pass(featsT, ws[:1], bs[:1], [], [], False)
    s1, t1 = _fold(st1, gs[0], bes[0], n)
    st2 = _mlp_pass(featsT, ws[:2], bs[:2], [s1], [t1], False)
    s2, t2 = _fold(st2, gs[1], bes[1], n)
    st3 = _mlp_pass(featsT, ws[:3], bs[:3], [s1, s2], [t1, t2], False)
    s3, t3 = _fold(st3, gs[2], bes[2], n)
    y4, st4 = _mlp_pass(featsT, ws, bs, [s1, s2, s3], [t1, t2, t3], True)
    s4, t4 = _fold(st4, g4, be4, n)

    agg = _sc_scatter_max(dst_ids, y4).reshape(NPAD, ACCW)

    w5t, b5, g5, be5 = params_out[0]
    w5 = jnp.zeros((ACCW, ACCW), f32).at[0:300, 0:300].set(w5t.T)
    b5p = jnp.zeros((1, ACCW), f32).at[:, 0:300].set(b5[None, :])
    g5p = jnp.zeros((1, ACCW), f32).at[:, 0:300].set(g5[None, :])
    be5p = jnp.zeros((1, ACCW), f32).at[:, 0:300].set(be5[None, :])
    out = _out_mlp(agg, s4[:, 0:ACCW], t4[:, 0:ACCW], w5, b5p, g5p, be5p)
    return out[:, 0:300]


# speedup vs baseline: 2.0827x; 1.7301x over previous
import functools

import jax
import jax.numpy as jnp
from jax import lax
from jax.experimental import pallas as pl
from jax.experimental.pallas import tpu as pltpu
from jax.experimental.pallas import tpu_sc as plsc

N_NODES = 10000
N_EDGES = 160000
NC, NS, L = 2, 16, 16
NW = NC * NS
EPS = 1e-5

FW = 8
EPTP = 5120
NE_PAD = NW * EPTP
GCH = 1280
NGCH = EPTP // GCH

NROW = 320
NPAD = NW * NROW
ACCW = 304
Y4W = 384
CH = 3200
NCHUNK = N_EDGES // CH
NVREG = CH // L
G = 32
MB = 192


def _gather_body(tab_a, tab_b, src_ids, dst_ids, out, ta, tb, idx_s, idx_d,
                 stage, sem):
    w = lax.axis_index("c") * NS + lax.axis_index("s")
    pltpu.sync_copy(tab_a, ta)
    pltpu.sync_copy(tab_b, tb)
    iota = lax.iota(jnp.int32, L)
    zero = jnp.zeros((L,), jnp.float32)

    for c in range(NGCH):
        base = w * EPTP + c * GCH
        pltpu.sync_copy(src_ids.at[pl.ds(base, GCH)], idx_s)
        pltpu.sync_copy(dst_ids.at[pl.ds(base, GCH)], idx_d)

        def vbody(i, _):
            e16 = i * L + iota
            isrc = idx_s[pl.ds(i * L, L)]
            idst = idx_d[pl.ds(i * L, L)]
            for j in range(4):
                va = plsc.load_gather(ta, [j * N_NODES + isrc])
                plsc.store_scatter(stage, [jnp.full((L,), j, jnp.int32), e16],
                                   va)
            for j in range(3):
                va = plsc.load_gather(ta, [(4 + j) * N_NODES + isrc])
                vb = plsc.load_gather(tb, [j * N_NODES + idst])
                plsc.store_scatter(stage,
                                   [jnp.full((L,), 4 + j, jnp.int32), e16],
                                   va - vb)
            plsc.store_scatter(stage, [jnp.full((L,), 7, jnp.int32), e16],
                               zero)
            return 0

        lax.fori_loop(0, GCH // L, vbody, 0)
        pltpu.sync_copy(stage, out.at[:, pl.ds(base, GCH)])


def _sc_gather(tab_a, tab_b, src_ids, dst_ids):
    mesh = plsc.VectorSubcoreMesh(core_axis_name="c", subcore_axis_name="s",
                                  num_cores=NC, num_subcores=NS)
    return pl.kernel(
        _gather_body,
        out_type=jax.ShapeDtypeStruct((FW, NE_PAD), jnp.float32),
        mesh=mesh,
        compiler_params=pltpu.CompilerParams(needs_layout_passes=False),
        scratch_types=[
            pltpu.VMEM((7 * N_NODES,), jnp.float32),
            pltpu.VMEM((3 * N_NODES,), jnp.float32),
            pltpu.VMEM((GCH,), jnp.int32),
            pltpu.VMEM((GCH,), jnp.int32),
            pltpu.VMEM((FW, GCH), jnp.float32),
            pltpu.SemaphoreType.DMA,
        ],
    )(tab_a, tab_b, src_ids, dst_ids)


def _scatter_body(dst_hbm, y4_hbm, out, acc, ids_v, midx, mnl, tmp, rows_v,
                  sem_i, sem_a, sem_b):
    w = lax.axis_index("c") * NS + lax.axis_index("s")
    lo = w * NROW
    iota = lax.iota(jnp.int32, L)
    neg1 = jnp.full((L,), -1.0, jnp.float32)
    dummy = jnp.full((L,), NROW, jnp.int32)
    zero_i = jnp.zeros((L,), jnp.int32)

    def ids_copy(c, slot):
        return pltpu.make_async_copy(dst_hbm.at[pl.ds(c * CH, CH)],
                                     ids_v.at[slot], sem_i)

    def grp_copy(g, slot):
        sem = sem_a if slot == 0 else sem_b
        return pltpu.make_async_copy(y4_hbm.at[midx.at[g]], rows_v.at[slot],
                                     sem)

    def init_body(i, _):
        acc[pl.ds(i * L, L)] = neg1
        return 0

    lax.fori_loop(0, (NROW + 1) * ACCW // L, init_body, 0)
    for q in range(MB // G):
        for r in range(G // L):
            midx[q, pl.ds(r * L, L)] = zero_i
    ids_copy(0, 0).start()

    def chunk_body(c, _):
        slot_c = c & 1
        ids_copy(c, slot_c).wait()

        @pl.when(c + 1 < NCHUNK)
        def _():
            ids_copy(c + 1, 1 - slot_c).start()

        for q in range(MB // L):
            mnl[pl.ds(q * L, L)] = dummy

        def scan_body(i, cnt_vec):
            v = ids_v[slot_c, pl.ds(i * L, L)]
            m = (v >= lo) & (v < lo + NROW)
            cs = plsc.cumsum(m.astype(jnp.int32))
            pos = cnt_vec + cs - 1
            eidx = c * CH + i * L + iota
            plsc.store_scatter(midx, [pos >> 5, pos & (G - 1)], eidx,
                               mask=m)
            plsc.store_scatter(mnl, [pos], v - lo, mask=m)
            tmp[...] = cs
            last = plsc.load_gather(tmp, [jnp.full((L,), L - 1, jnp.int32)])
            return cnt_vec + last

        cnt_vec = lax.fori_loop(0, NVREG, scan_body, jnp.zeros((L,), jnp.int32))
        cnt = jnp.max(cnt_vec)
        ngroups = (cnt + (G - 1)) // G

        @pl.when(ngroups > 0)
        def _():
            grp_copy(0, 0).start()

        def flush_body(g, _):
            gslot = g & 1
            more = g + 1 < ngroups

            @pl.when(jnp.logical_and(more, gslot == 0))
            def _():
                grp_copy(g + 1, 1).start()

            @pl.when(jnp.logical_and(more, gslot == 1))
            def _():
                grp_copy(g + 1, 0).start()

            @pl.when(gslot == 0)
            def _():
                grp_copy(g, 0).wait()

            @pl.when(gslot == 1)
            def _():
                grp_copy(g, 1).wait()

            def rmw_body(eo, _):
                nlv = mnl[pl.ds(g * G + eo * 4, L)]
                for k in range(4):
                    nl = nlv[k]
                    rr = eo * 4 + k
                    base = nl * ACCW
                    for j in range(ACCW // L):
                        a = acc[pl.ds(base + j * L, L)]
                        b = rows_v[gslot, rr, pl.ds(j * L, L)]
                        acc[pl.ds(base + j * L, L)] = jnp.maximum(a, b)
                return 0

            lax.fori_loop(0, G // 4, rmw_body, 0)
            return 0

        lax.fori_loop(0, ngroups, flush_body, 0)
        return 0

    lax.fori_loop(0, NCHUNK, chunk_body, 0)
    pltpu.sync_copy(acc.at[pl.ds(0, NROW * ACCW)],
                    out.at[pl.ds(lo * ACCW, NROW * ACCW)])


def _sc_scatter_max(dst_ids, y4):
    mesh = plsc.VectorSubcoreMesh(core_axis_name="c", subcore_axis_name="s",
                                  num_cores=NC, num_subcores=NS)
    return pl.kernel(
        _scatter_body,
        out_type=jax.ShapeDtypeStruct((NPAD * ACCW,), jnp.float32),
        mesh=mesh,
        compiler_params=pltpu.CompilerParams(needs_layout_passes=False),
        scratch_types=[
            pltpu.VMEM(((NROW + 1) * ACCW,), jnp.float32),
            pltpu.VMEM((2, CH), jnp.int32),
            pltpu.VMEM((MB // G, G), jnp.int32),
            pltpu.VMEM((MB + L,), jnp.int32),
            pltpu.VMEM((L,), jnp.int32),
            pltpu.VMEM((2, G, Y4W), jnp.float32),
            pltpu.SemaphoreType.DMA,
            pltpu.SemaphoreType.DMA,
            pltpu.SemaphoreType.DMA,
        ],
    )(dst_ids, y4)


BLK = 6400
NBLK = N_EDGES // BLK


def _pass_body(nlayers, write_y, *refs):
    k = nlayers
    x_ref = refs[0]
    w_refs = refs[1:1 + k]
    b_refs = refs[1 + k:1 + 2 * k]
    s_refs = refs[1 + 2 * k:3 * k]
    t_refs = refs[3 * k:4 * k - 1]
    if write_y:
        y_ref = refs[4 * k - 1]
        stats_ref = refs[4 * k]
    else:
        stats_ref = refs[4 * k - 1]
    i = pl.program_id(0)
    xt = x_ref[...]
    a = lax.dot_general(xt, w_refs[0][...], (((0,), (0,)), ((), ())),
                        preferred_element_type=jnp.float32)
    a = jnp.maximum(a + b_refs[0][...], 0.0)
    for li in range(1, k):
        h = a * s_refs[li - 1][...] + t_refs[li - 1][...]
        a = jnp.dot(h, w_refs[li][...], preferred_element_type=jnp.float32)
        a = jnp.maximum(a + b_refs[li][...], 0.0)
    dout = a.shape[1]
    ssum = jnp.sum(a, axis=0, keepdims=True)
    ssq = jnp.sum(a * a, axis=0, keepdims=True)
    st = jnp.concatenate(
        [ssum, ssq, jnp.zeros((6, dout), jnp.float32)], axis=0)

    @pl.when(i == 0)
    def _():
        stats_ref[...] = jnp.zeros_like(stats_ref)

    stats_ref[...] += st
    if write_y:
        y_ref[...] = a


def _mlp_pass(x0t, ws, bs, ss, ts, write_y):
    k = len(ws)
    dout = ws[-1].shape[1]
    full = lambda shape: pl.BlockSpec(shape, lambda i: (0, 0))
    in_specs = [pl.BlockSpec((FW, BLK), lambda i: (0, i))]
    in_specs += [full(w.shape) for w in ws]
    in_specs += [full((1, b.shape[1])) for b in bs]
    in_specs += [full((1, s.shape[1])) for s in ss]
    in_specs += [full((1, t.shape[1])) for t in ts]
    out_shapes = []
    out_specs = []
    if write_y:
        out_shapes.append(jax.ShapeDtypeStruct((N_EDGES, dout), jnp.float32))
        out_specs.append(pl.BlockSpec((BLK, dout), lambda i: (i, 0)))
    out_shapes.append(jax.ShapeDtypeStruct((8, dout), jnp.float32))
    out_specs.append(full((8, dout)))
    out = pl.pallas_call(
        functools.partial(_pass_body, k, write_y),
        grid=(NBLK,),
        in_specs=in_specs,
        out_specs=out_specs,
        out_shape=out_shapes,
    )(x0t, *ws, *bs, *ss, *ts)
    return tuple(out) if write_y else out[0]


def _out_body(x_ref, s4_ref, t4_ref, w_ref, b_ref, g_ref, be_ref, o_ref):
    x = x_ref[pl.ds(0, N_NODES), :]
    xb = jnp.where(x < 0.0, 0.0, x * s4_ref[...] + t4_ref[...])
    z = jnp.dot(xb, w_ref[...], preferred_element_type=jnp.float32)
    z = jnp.maximum(z + b_ref[...], 0.0)
    n = z.shape[0]
    mean = jnp.sum(z, axis=0, keepdims=True) / n
    var = jnp.sum(z * z, axis=0, keepdims=True) / n - mean * mean
    inv = g_ref[...] * lax.rsqrt(var + EPS)
    o_ref[...] = (z - mean) * inv + be_ref[...]


def _out_mlp(xagg, s4, t4, w5, b5, g5, be5):
    return pl.pallas_call(
        _out_body,
        out_shape=jax.ShapeDtypeStruct((N_NODES, ACCW), jnp.float32),
    )(xagg, s4, t4, w5, b5, g5, be5)


def _fold(stats, gamma, beta, n):
    mean = stats[0] / n
    var = stats[1] / n - mean * mean
    s = gamma * lax.rsqrt(var + EPS)
    t = beta - mean * s
    return s[None, :], t[None, :]


def kernel(last_coors, last_features, current_coors, edge, params_in,
           params_out):
    f32 = jnp.float32
    tab_a = jnp.concatenate([last_features.T, last_coors.T],
                            axis=0).reshape(-1)
    tab_b = current_coors.T.reshape(-1)
    dst_ids = edge[0]
    pad = jnp.zeros((NE_PAD - N_EDGES,), jnp.int32)
    src_pad = jnp.concatenate([edge[1], pad])
    dst_pad = jnp.concatenate([dst_ids, pad])

    featsT = _sc_gather(tab_a, tab_b, src_pad, dst_pad)

    ws, bs, gs, bes = [], [], [], []
    for (wt, b, g, be) in params_in:
        ws.append(wt.T)
        bs.append(b[None, :])
        gs.append(g)
        bes.append(be)
    w1 = jnp.zeros((FW, ws[0].shape[1]), f32).at[0:7, :].set(ws[0])
    w4 = jnp.zeros((ws[3].shape[0], Y4W), f32).at[:, 0:300].set(ws[3])
    b4 = jnp.zeros((1, Y4W), f32).at[:, 0:300].set(bs[3])
    g4 = jnp.zeros((Y4W,), f32).at[0:300].set(gs[3])
    be4 = jnp.zeros((Y4W,), f32).at[0:300].set(bes[3])
    ws = [w1, ws[1], ws[2], w4]
    bs = [bs[0], bs[1], bs[2], b4]

    n = float(N_EDGES)
    st1 = _mlp_pass(featsT, ws[:1], bs[:1], [], [], False)
    s1, t1 = _fold(st1, gs[0], bes[0], n)
    st2 = _mlp_pass(featsT, ws[:2], bs[:2], [s1], [t1], False)
    s2, t2 = _fold(st2, gs[1], bes[1], n)
    st3 = _mlp_pass(featsT, ws[:3], bs[:3], [s1, s2], [t1, t2], False)
    s3, t3 = _fold(st3, gs[2], bes[2], n)
    y4, st4 = _mlp_pass(featsT, ws, bs, [s1, s2, s3], [t1, t2, t3], True)
    s4, t4 = _fold(st4, g4, be4, n)

    agg = _sc_scatter_max(dst_ids, y4).reshape(NPAD, ACCW)

    w5t, b5, g5, be5 = params_out[0]
    w5 = jnp.zeros((ACCW, ACCW), f32).at[0:300, 0:300].set(w5t.T)
    b5p = jnp.zeros((1, ACCW), f32).at[:, 0:300].set(b5[None, :])
    g5p = jnp.zeros((1, ACCW), f32).at[:, 0:300].set(g5[None, :])
    be5p = jnp.zeros((1, ACCW), f32).at[:, 0:300].set(be5[None, :])
    out = _out_mlp(agg, s4[:, 0:ACCW], t4[:, 0:ACCW], w5, b5p, g5p, be5p)
    return out[:, 0:300]


# speedup vs baseline: 2.3629x; 1.1346x over previous
import functools

import jax
import jax.numpy as jnp
from jax import lax
from jax.experimental import pallas as pl
from jax.experimental.pallas import tpu as pltpu
from jax.experimental.pallas import tpu_sc as plsc

N_NODES = 10000
N_EDGES = 160000
NC, NS, L = 2, 16, 16
NW = NC * NS
EPS = 1e-5

FW = 8
EPTP = 5120
NE_PAD = NW * EPTP
GCH = 1280
NGCH = EPTP // GCH

NROW = 320
NPAD = NW * NROW
ACCW = 304
Y4W = 384
CH = 6400
NCHUNK = N_EDGES // CH
NVREG = CH // L
G = 32
MB = 320


def _gather_body(tab_a, tab_b, src_ids, dst_ids, out, ta, tb, idx_s, idx_d,
                 stage, sem):
    w = lax.axis_index("c") * NS + lax.axis_index("s")
    pltpu.sync_copy(tab_a, ta)
    pltpu.sync_copy(tab_b, tb)
    iota = lax.iota(jnp.int32, L)
    zero = jnp.zeros((L,), jnp.float32)

    for c in range(NGCH):
        base = w * EPTP + c * GCH
        pltpu.sync_copy(src_ids.at[pl.ds(base, GCH)], idx_s)
        pltpu.sync_copy(dst_ids.at[pl.ds(base, GCH)], idx_d)

        def vbody(i, _):
            e16 = i * L + iota
            isrc = idx_s[pl.ds(i * L, L)]
            idst = idx_d[pl.ds(i * L, L)]
            for j in range(4):
                va = plsc.load_gather(ta, [j * N_NODES + isrc])
                plsc.store_scatter(stage, [jnp.full((L,), j, jnp.int32), e16],
                                   va)
            for j in range(3):
                va = plsc.load_gather(ta, [(4 + j) * N_NODES + isrc])
                vb = plsc.load_gather(tb, [j * N_NODES + idst])
                plsc.store_scatter(stage,
                                   [jnp.full((L,), 4 + j, jnp.int32), e16],
                                   va - vb)
            plsc.store_scatter(stage, [jnp.full((L,), 7, jnp.int32), e16],
                               zero)
            return 0

        lax.fori_loop(0, GCH // L, vbody, 0)
        pltpu.sync_copy(stage, out.at[:, pl.ds(base, GCH)])


def _sc_gather(tab_a, tab_b, src_ids, dst_ids):
    mesh = plsc.VectorSubcoreMesh(core_axis_name="c", subcore_axis_name="s",
                                  num_cores=NC, num_subcores=NS)
    return pl.kernel(
        _gather_body,
        out_type=jax.ShapeDtypeStruct((FW, NE_PAD), jnp.float32),
        mesh=mesh,
        compiler_params=pltpu.CompilerParams(needs_layout_passes=False),
        scratch_types=[
            pltpu.VMEM((7 * N_NODES,), jnp.float32),
            pltpu.VMEM((3 * N_NODES,), jnp.float32),
            pltpu.VMEM((GCH,), jnp.int32),
            pltpu.VMEM((GCH,), jnp.int32),
            pltpu.VMEM((FW, GCH), jnp.float32),
            pltpu.SemaphoreType.DMA,
        ],
    )(tab_a, tab_b, src_ids, dst_ids)


def _scatter_body(dst_hbm, y4_hbm, out, acc, ids_v, midx, mnl, rows_v,
                  sem_i, sem_a, sem_b):
    w = lax.axis_index("c") * NS + lax.axis_index("s")
    lo = w * NROW
    iota = lax.iota(jnp.int32, L)
    neg1 = jnp.full((L,), -1.0, jnp.float32)
    dummy = jnp.full((L,), NROW, jnp.int32)
    zero_i = jnp.zeros((L,), jnp.int32)

    def ids_copy(c):
        return pltpu.make_async_copy(dst_hbm.at[pl.ds(c * CH, CH)],
                                     ids_v, sem_i)

    def grp_copy(g, slot):
        sem = sem_a if slot == 0 else sem_b
        return pltpu.make_async_copy(y4_hbm.at[midx.at[g]], rows_v.at[slot],
                                     sem)

    def init_body(i, _):
        acc[pl.ds(i * L, L)] = neg1
        return 0

    lax.fori_loop(0, (NROW + 1) * ACCW // L, init_body, 0)
    for q in range(MB // G):
        for r in range(G // L):
            midx[q, pl.ds(r * L, L)] = zero_i
    def chunk_body(c, _):
        ids_copy(c).start()
        ids_copy(c).wait()

        for q in range(MB // L):
            mnl[pl.ds(q * L, L)] = dummy

        def scan_body(i, cnt):
            v = ids_v[pl.ds(i * L, L)]
            m = (v >= lo) & (v < lo + NROW)
            cs = plsc.cumsum(m.astype(jnp.int32))
            pos = cnt + cs - 1
            eidx = c * CH + i * L + iota
            plsc.store_scatter(midx, [pos >> 5, pos & (G - 1)], eidx,
                               mask=m)
            plsc.store_scatter(mnl, [pos], v - lo, mask=m)
            return cnt + cs[L - 1]

        cnt = lax.fori_loop(0, NVREG, scan_body, jnp.int32(0))
        ngroups = (cnt + (G - 1)) // G

        @pl.when(ngroups > 0)
        def _():
            grp_copy(0, 0).start()

        def flush_body(g, _):
            gslot = g & 1
            more = g + 1 < ngroups

            @pl.when(jnp.logical_and(more, gslot == 0))
            def _():
                grp_copy(g + 1, 1).start()

            @pl.when(jnp.logical_and(more, gslot == 1))
            def _():
                grp_copy(g + 1, 0).start()

            @pl.when(gslot == 0)
            def _():
                grp_copy(g, 0).wait()

            @pl.when(gslot == 1)
            def _():
                grp_copy(g, 1).wait()

            def rmw_body(eo, _):
                nlv = mnl[pl.ds(g * G + eo * 4, L)]
                for k in range(4):
                    nl = nlv[k]
                    rr = eo * 4 + k
                    base = nl * ACCW
                    for j in range(ACCW // L):
                        a = acc[pl.ds(base + j * L, L)]
                        b = rows_v[gslot, rr, pl.ds(j * L, L)]
                        acc[pl.ds(base + j * L, L)] = jnp.maximum(a, b)
                return 0

            lax.fori_loop(0, G // 4, rmw_body, 0)
            return 0

        lax.fori_loop(0, ngroups, flush_body, 0)
        return 0

    lax.fori_loop(0, NCHUNK, chunk_body, 0)
    pltpu.sync_copy(acc.at[pl.ds(0, NROW * ACCW)],
                    out.at[pl.ds(lo * ACCW, NROW * ACCW)])


def _sc_scatter_max(dst_ids, y4):
    mesh = plsc.VectorSubcoreMesh(core_axis_name="c", subcore_axis_name="s",
                                  num_cores=NC, num_subcores=NS)
    return pl.kernel(
        _scatter_body,
        out_type=jax.ShapeDtypeStruct((NPAD * ACCW,), jnp.float32),
        mesh=mesh,
        compiler_params=pltpu.CompilerParams(needs_layout_passes=False),
        scratch_types=[
            pltpu.VMEM(((NROW + 1) * ACCW,), jnp.float32),
            pltpu.VMEM((CH,), jnp.int32),
            pltpu.VMEM((MB // G, G), jnp.int32),
            pltpu.VMEM((MB + L,), jnp.int32),
            pltpu.VMEM((2, G, Y4W), jnp.float32),
            pltpu.SemaphoreType.DMA,
            pltpu.SemaphoreType.DMA,
            pltpu.SemaphoreType.DMA,
        ],
    )(dst_ids, y4)


BLK = 6400
NBLK = N_EDGES // BLK


def _pass_body(nlayers, write_y, *refs):
    k = nlayers
    x_ref = refs[0]
    w_refs = refs[1:1 + k]
    b_refs = refs[1 + k:1 + 2 * k]
    s_refs = refs[1 + 2 * k:3 * k]
    t_refs = refs[3 * k:4 * k - 1]
    if write_y:
        y_ref = refs[4 * k - 1]
        stats_ref = refs[4 * k]
    else:
        stats_ref = refs[4 * k - 1]
    i = pl.program_id(0)
    xt = x_ref[...]
    a = lax.dot_general(xt, w_refs[0][...], (((0,), (0,)), ((), ())),
                        preferred_element_type=jnp.float32)
    a = jnp.maximum(a + b_refs[0][...], 0.0)
    for li in range(1, k):
        h = a * s_refs[li - 1][...] + t_refs[li - 1][...]
        a = jnp.dot(h, w_refs[li][...], preferred_element_type=jnp.float32)
        a = jnp.maximum(a + b_refs[li][...], 0.0)
    dout = a.shape[1]
    ssum = jnp.sum(a, axis=0, keepdims=True)
    ssq = jnp.sum(a * a, axis=0, keepdims=True)
    st = jnp.concatenate(
        [ssum, ssq, jnp.zeros((6, dout), jnp.float32)], axis=0)

    @pl.when(i == 0)
    def _():
        stats_ref[...] = jnp.zeros_like(stats_ref)

    stats_ref[...] += st
    if write_y:
        y_ref[...] = a


def _mlp_pass(x0t, ws, bs, ss, ts, write_y):
    k = len(ws)
    dout = ws[-1].shape[1]
    full = lambda shape: pl.BlockSpec(shape, lambda i: (0, 0))
    in_specs = [pl.BlockSpec((FW, BLK), lambda i: (0, i))]
    in_specs += [full(w.shape) for w in ws]
    in_specs += [full((1, b.shape[1])) for b in bs]
    in_specs += [full((1, s.shape[1])) for s in ss]
    in_specs += [full((1, t.shape[1])) for t in ts]
    out_shapes = []
    out_specs = []
    if write_y:
        out_shapes.append(jax.ShapeDtypeStruct((N_EDGES, dout), jnp.float32))
        out_specs.append(pl.BlockSpec((BLK, dout), lambda i: (i, 0)))
    out_shapes.append(jax.ShapeDtypeStruct((8, dout), jnp.float32))
    out_specs.append(full((8, dout)))
    out = pl.pallas_call(
        functools.partial(_pass_body, k, write_y),
        grid=(NBLK,),
        in_specs=in_specs,
        out_specs=out_specs,
        out_shape=out_shapes,
    )(x0t, *ws, *bs, *ss, *ts)
    return tuple(out) if write_y else out[0]


def _out_body(x_ref, s4_ref, t4_ref, w_ref, b_ref, g_ref, be_ref, o_ref):
    x = x_ref[pl.ds(0, N_NODES), :]
    xb = jnp.where(x < 0.0, 0.0, x * s4_ref[...] + t4_ref[...])
    z = jnp.dot(xb, w_ref[...], preferred_element_type=jnp.float32)
    z = jnp.maximum(z + b_ref[...], 0.0)
    n = z.shape[0]
    mean = jnp.sum(z, axis=0, keepdims=True) / n
    var = jnp.sum(z * z, axis=0, keepdims=True) / n - mean * mean
    inv = g_ref[...] * lax.rsqrt(var + EPS)
    o_ref[...] = ((z - mean) * inv + be_ref[...])[:, 0:300]


def _out_mlp(xagg, s4, t4, w5, b5, g5, be5):
    return pl.pallas_call(
        _out_body,
        out_shape=jax.ShapeDtypeStruct((N_NODES, 300), jnp.float32),
    )(xagg, s4, t4, w5, b5, g5, be5)


def _fold(stats, gamma, beta, n):
    mean = stats[0] / n
    var = stats[1] / n - mean * mean
    s = gamma * lax.rsqrt(var + EPS)
    t = beta - mean * s
    return s[None, :], t[None, :]


def kernel(last_coors, last_features, current_coors, edge, params_in,
           params_out):
    f32 = jnp.float32
    tab_a = jnp.concatenate([last_features.T, last_coors.T],
                            axis=0).reshape(-1)
    tab_b = current_coors.T.reshape(-1)
    dst_ids = edge[0]
    pad = jnp.zeros((NE_PAD - N_EDGES,), jnp.int32)
    src_pad = jnp.concatenate([edge[1], pad])
    dst_pad = jnp.concatenate([dst_ids, pad])

    featsT = _sc_gather(tab_a, tab_b, src_pad, dst_pad)

    ws, bs, gs, bes = [], [], [], []
    for (wt, b, g, be) in params_in:
        ws.append(wt.T)
        bs.append(b[None, :])
        gs.append(g)
        bes.append(be)
    w1 = jnp.zeros((FW, ws[0].shape[1]), f32).at[0:7, :].set(ws[0])
    w4 = jnp.zeros((ws[3].shape[0], Y4W), f32).at[:, 0:300].set(ws[3])
    b4 = jnp.zeros((1, Y4W), f32).at[:, 0:300].set(bs[3])
    g4 = jnp.zeros((Y4W,), f32).at[0:300].set(gs[3])
    be4 = jnp.zeros((Y4W,), f32).at[0:300].set(bes[3])
    ws = [w1, ws[1], ws[2], w4]
    bs = [bs[0], bs[1], bs[2], b4]

    n = float(N_EDGES)
    st1 = _mlp_pass(featsT, ws[:1], bs[:1], [], [], False)
    s1, t1 = _fold(st1, gs[0], bes[0], n)
    st2 = _mlp_pass(featsT, ws[:2], bs[:2], [s1], [t1], False)
    s2, t2 = _fold(st2, gs[1], bes[1], n)
    st3 = _mlp_pass(featsT, ws[:3], bs[:3], [s1, s2], [t1, t2], False)
    s3, t3 = _fold(st3, gs[2], bes[2], n)
    y4, st4 = _mlp_pass(featsT, ws, bs, [s1, s2, s3], [t1, t2, t3], True)
    s4, t4 = _fold(st4, g4, be4, n)

    agg = _sc_scatter_max(dst_ids, y4).reshape(NPAD, ACCW)

    w5t, b5, g5, be5 = params_out[0]
    w5 = jnp.zeros((ACCW, ACCW), f32).at[0:300, 0:300].set(w5t.T)
    b5p = jnp.zeros((1, ACCW), f32).at[:, 0:300].set(b5[None, :])
    g5p = jnp.zeros((1, ACCW), f32).at[:, 0:300].set(g5[None, :])
    be5p = jnp.zeros((1, ACCW), f32).at[:, 0:300].set(be5[None, :])
    return _out_mlp(agg, s4[:, 0:ACCW], t4[:, 0:ACCW], w5, b5p, g5p, be5p)


# speedup vs baseline: 2.4223x; 1.0251x over previous
import functools

import jax
import jax.numpy as jnp
from jax import lax
from jax.experimental import pallas as pl
from jax.experimental.pallas import tpu as pltpu
from jax.experimental.pallas import tpu_sc as plsc

N_NODES = 10000
N_EDGES = 160000
NC, NS, L = 2, 16, 16
NW = NC * NS
EPS = 1e-5

FW = 8
EPTP = 5120
NE_PAD = NW * EPTP
GCH = 1280
NGCH = EPTP // GCH

NROW = 320
NPAD = NW * NROW
ACCW = 304
Y4W = 384
CH = 6400
NCHUNK = N_EDGES // CH
NVREG = CH // L
G = 32
MB = 320


def _gather_body(tab_a, tab_b, src_ids, dst_ids, out, ta, tb, idx_s, idx_d,
                 stage, sem):
    w = lax.axis_index("c") * NS + lax.axis_index("s")
    pltpu.sync_copy(tab_a, ta)
    pltpu.sync_copy(tab_b, tb)
    iota = lax.iota(jnp.int32, L)
    zero = jnp.zeros((L,), jnp.float32)

    for c in range(NGCH):
        base = w * EPTP + c * GCH
        pltpu.sync_copy(src_ids.at[pl.ds(base, GCH)], idx_s)
        pltpu.sync_copy(dst_ids.at[pl.ds(base, GCH)], idx_d)

        def vbody(i, _):
            e16 = i * L + iota
            isrc = idx_s[pl.ds(i * L, L)]
            idst = idx_d[pl.ds(i * L, L)]
            for j in range(4):
                va = plsc.load_gather(ta, [j * N_NODES + isrc])
                plsc.store_scatter(stage, [jnp.full((L,), j, jnp.int32), e16],
                                   va)
            for j in range(3):
                va = plsc.load_gather(ta, [(4 + j) * N_NODES + isrc])
                vb = plsc.load_gather(tb, [j * N_NODES + idst])
                plsc.store_scatter(stage,
                                   [jnp.full((L,), 4 + j, jnp.int32), e16],
                                   va - vb)
            plsc.store_scatter(stage, [jnp.full((L,), 7, jnp.int32), e16],
                               zero)
            return 0

        lax.fori_loop(0, GCH // L, vbody, 0)
        pltpu.sync_copy(stage, out.at[:, pl.ds(base, GCH)])


def _sc_gather(tab_a, tab_b, src_ids, dst_ids):
    mesh = plsc.VectorSubcoreMesh(core_axis_name="c", subcore_axis_name="s",
                                  num_cores=NC, num_subcores=NS)
    return pl.kernel(
        _gather_body,
        out_type=jax.ShapeDtypeStruct((FW, NE_PAD), jnp.float32),
        mesh=mesh,
        compiler_params=pltpu.CompilerParams(needs_layout_passes=False),
        scratch_types=[
            pltpu.VMEM((7 * N_NODES,), jnp.float32),
            pltpu.VMEM((3 * N_NODES,), jnp.float32),
            pltpu.VMEM((GCH,), jnp.int32),
            pltpu.VMEM((GCH,), jnp.int32),
            pltpu.VMEM((FW, GCH), jnp.float32),
            pltpu.SemaphoreType.DMA,
        ],
    )(tab_a, tab_b, src_ids, dst_ids)


def _scatter_body(dst_hbm, y4_hbm, out, acc, ids_v, midx, mnl, rows_v,
                  sem_i, sem_a, sem_b):
    w = lax.axis_index("c") * NS + lax.axis_index("s")
    lo = w * NROW
    iota = lax.iota(jnp.int32, L)
    neg1 = jnp.full((L,), -1.0, jnp.float32)
    dummy = jnp.full((L,), NROW, jnp.int32)
    zero_i = jnp.zeros((L,), jnp.int32)

    def ids_copy(c):
        return pltpu.make_async_copy(dst_hbm.at[pl.ds(c * CH, CH)],
                                     ids_v, sem_i)

    def grp_copy(g, slot):
        sem = sem_a if slot == 0 else sem_b
        return pltpu.make_async_copy(y4_hbm.at[midx.at[g]], rows_v.at[slot],
                                     sem)

    def init_body(i, _):
        acc[pl.ds(i * L, L)] = neg1
        return 0

    lax.fori_loop(0, (NROW + 1) * ACCW // L, init_body, 0)
    for q in range(MB // G):
        for r in range(G // L):
            midx[q, pl.ds(r * L, L)] = zero_i
    ids_copy(0).start()

    def chunk_body(c, _):
        ids_copy(c).wait()

        for q in range(MB // L):
            mnl[pl.ds(q * L, L)] = dummy

        def scan_body(i, cnt):
            v = ids_v[pl.ds(i * L, L)]
            m = (v >= lo) & (v < lo + NROW)
            cs = plsc.cumsum(m.astype(jnp.int32))
            pos = cnt + cs - 1
            eidx = c * CH + i * L + iota
            plsc.store_scatter(midx, [pos >> 5, pos & (G - 1)], eidx,
                               mask=m)
            plsc.store_scatter(mnl, [pos], v - lo, mask=m)
            return cnt + cs[L - 1]

        cnt = lax.fori_loop(0, NVREG, scan_body, jnp.int32(0))

        @pl.when(c + 1 < NCHUNK)
        def _():
            ids_copy(c + 1).start()

        ngroups = (cnt + (G - 1)) // G

        @pl.when(ngroups > 0)
        def _():
            grp_copy(0, 0).start()

        def flush_body(g, _):
            gslot = g & 1
            more = g + 1 < ngroups

            @pl.when(jnp.logical_and(more, gslot == 0))
            def _():
                grp_copy(g + 1, 1).start()

            @pl.when(jnp.logical_and(more, gslot == 1))
            def _():
                grp_copy(g + 1, 0).start()

            @pl.when(gslot == 0)
            def _():
                grp_copy(g, 0).wait()

            @pl.when(gslot == 1)
            def _():
                grp_copy(g, 1).wait()

            def rmw_body(eo, _):
                nlv = mnl[pl.ds(g * G + eo * 8, L)]
                for k in range(8):
                    nl = nlv[k]
                    rr = eo * 8 + k
                    base = nl * ACCW
                    for j in range(ACCW // L):
                        a = acc[pl.ds(base + j * L, L)]
                        b = rows_v[gslot, rr, pl.ds(j * L, L)]
                        acc[pl.ds(base + j * L, L)] = jnp.maximum(a, b)
                return 0

            lax.fori_loop(0, G // 8, rmw_body, 0)
            return 0

        lax.fori_loop(0, ngroups, flush_body, 0)
        return 0

    lax.fori_loop(0, NCHUNK, chunk_body, 0)
    pltpu.sync_copy(acc.at[pl.ds(0, NROW * ACCW)],
                    out.at[pl.ds(lo * ACCW, NROW * ACCW)])


def _sc_scatter_max(dst_ids, y4):
    mesh = plsc.VectorSubcoreMesh(core_axis_name="c", subcore_axis_name="s",
                                  num_cores=NC, num_subcores=NS)
    return pl.kernel(
        _scatter_body,
        out_type=jax.ShapeDtypeStruct((NPAD * ACCW,), jnp.float32),
        mesh=mesh,
        compiler_params=pltpu.CompilerParams(needs_layout_passes=False),
        scratch_types=[
            pltpu.VMEM(((NROW + 1) * ACCW,), jnp.float32),
            pltpu.VMEM((CH,), jnp.int32),
            pltpu.VMEM((MB // G, G), jnp.int32),
            pltpu.VMEM((MB + L,), jnp.int32),
            pltpu.VMEM((2, G, Y4W), jnp.float32),
            pltpu.SemaphoreType.DMA,
            pltpu.SemaphoreType.DMA,
            pltpu.SemaphoreType.DMA,
        ],
    )(dst_ids, y4)


BLK = 6400
NBLK = N_EDGES // BLK


def _pass_body(nlayers, write_y, *refs):
    k = nlayers
    x_ref = refs[0]
    w_refs = refs[1:1 + k]
    b_refs = refs[1 + k:1 + 2 * k]
    s_refs = refs[1 + 2 * k:3 * k]
    t_refs = refs[3 * k:4 * k - 1]
    if write_y:
        y_ref = refs[4 * k - 1]
        stats_ref = refs[4 * k]
    else:
        stats_ref = refs[4 * k - 1]
    i = pl.program_id(0)
    xt = x_ref[...]
    a = lax.dot_general(xt, w_refs[0][...], (((0,), (0,)), ((), ())),
                        preferred_element_type=jnp.float32)
    a = jnp.maximum(a + b_refs[0][...], 0.0)
    for li in range(1, k):
        h = a * s_refs[li - 1][...] + t_refs[li - 1][...]
        a = jnp.dot(h, w_refs[li][...], preferred_element_type=jnp.float32)
        a = jnp.maximum(a + b_refs[li][...], 0.0)
    dout = a.shape[1]
    ssum = jnp.sum(a, axis=0, keepdims=True)
    ssq = jnp.sum(a * a, axis=0, keepdims=True)
    st = jnp.concatenate(
        [ssum, ssq, jnp.zeros((6, dout), jnp.float32)], axis=0)

    @pl.when(i == 0)
    def _():
        stats_ref[...] = jnp.zeros_like(stats_ref)

    stats_ref[...] += st
    if write_y:
        y_ref[...] = a


def _mlp_pass(x0t, ws, bs, ss, ts, write_y):
    k = len(ws)
    dout = ws[-1].shape[1]
    full = lambda shape: pl.BlockSpec(shape, lambda i: (0, 0))
    in_specs = [pl.BlockSpec((FW, BLK), lambda i: (0, i))]
    in_specs += [full(w.shape) for w in ws]
    in_specs += [full((1, b.shape[1])) for b in bs]
    in_specs += [full((1, s.shape[1])) for s in ss]
    in_specs += [full((1, t.shape[1])) for t in ts]
    out_shapes = []
    out_specs = []
    if write_y:
        out_shapes.append(jax.ShapeDtypeStruct((N_EDGES, dout), jnp.float32))
        out_specs.append(pl.BlockSpec((BLK, dout), lambda i: (i, 0)))
    out_shapes.append(jax.ShapeDtypeStruct((8, dout), jnp.float32))
    out_specs.append(full((8, dout)))
    out = pl.pallas_call(
        functools.partial(_pass_body, k, write_y),
        grid=(NBLK,),
        in_specs=in_specs,
        out_specs=out_specs,
        out_shape=out_shapes,
    )(x0t, *ws, *bs, *ss, *ts)
    return tuple(out) if write_y else out[0]


def _out_body(x_ref, s4_ref, t4_ref, w_ref, b_ref, g_ref, be_ref, o_ref):
    x = x_ref[pl.ds(0, N_NODES), :]
    xb = jnp.where(x < 0.0, 0.0, x * s4_ref[...] + t4_ref[...])
    z = jnp.dot(xb, w_ref[...], preferred_element_type=jnp.float32)
    z = jnp.maximum(z + b_ref[...], 0.0)
    n = z.shape[0]
    mean = jnp.sum(z, axis=0, keepdims=True) / n
    var = jnp.sum(z * z, axis=0, keepdims=True) / n - mean * mean
    inv = g_ref[...] * lax.rsqrt(var + EPS)
    o_ref[...] = ((z - mean) * inv + be_ref[...])[:, 0:300]


def _out_mlp(xagg, s4, t4, w5, b5, g5, be5):
    return pl.pallas_call(
        _out_body,
        out_shape=jax.ShapeDtypeStruct((N_NODES, 300), jnp.float32),
    )(xagg, s4, t4, w5, b5, g5, be5)


def _fold(stats, gamma, beta, n):
    mean = stats[0] / n
    var = stats[1] / n - mean * mean
    s = gamma * lax.rsqrt(var + EPS)
    t = beta - mean * s
    return s[None, :], t[None, :]


def kernel(last_coors, last_features, current_coors, edge, params_in,
           params_out):
    f32 = jnp.float32
    tab_a = jnp.concatenate([last_features.T, last_coors.T],
                            axis=0).reshape(-1)
    tab_b = current_coors.T.reshape(-1)
    dst_ids = edge[0]
    pad = jnp.zeros((NE_PAD - N_EDGES,), jnp.int32)
    src_pad = jnp.concatenate([edge[1], pad])
    dst_pad = jnp.concatenate([dst_ids, pad])

    featsT = _sc_gather(tab_a, tab_b, src_pad, dst_pad)

    ws, bs, gs, bes = [], [], [], []
    for (wt, b, g, be) in params_in:
        ws.append(wt.T)
        bs.append(b[None, :])
        gs.append(g)
        bes.append(be)
    w1 = jnp.zeros((FW, ws[0].shape[1]), f32).at[0:7, :].set(ws[0])
    w4 = jnp.zeros((ws[3].shape[0], Y4W), f32).at[:, 0:300].set(ws[3])
    b4 = jnp.zeros((1, Y4W), f32).at[:, 0:300].set(bs[3])
    g4 = jnp.zeros((Y4W,), f32).at[0:300].set(gs[3])
    be4 = jnp.zeros((Y4W,), f32).at[0:300].set(bes[3])
    ws = [w1, ws[1], ws[2], w4]
    bs = [bs[0], bs[1], bs[2], b4]

    n = float(N_EDGES)
    st1 = _mlp_pass(featsT, ws[:1], bs[:1], [], [], False)
    s1, t1 = _fold(st1, gs[0], bes[0], n)
    st2 = _mlp_pass(featsT, ws[:2], bs[:2], [s1], [t1], False)
    s2, t2 = _fold(st2, gs[1], bes[1], n)
    st3 = _mlp_pass(featsT, ws[:3], bs[:3], [s1, s2], [t1, t2], False)
    s3, t3 = _fold(st3, gs[2], bes[2], n)
    y4, st4 = _mlp_pass(featsT, ws, bs, [s1, s2, s3], [t1, t2, t3], True)
    s4, t4 = _fold(st4, g4, be4, n)

    agg = _sc_scatter_max(dst_ids, y4).reshape(NPAD, ACCW)

    w5t, b5, g5, be5 = params_out[0]
    w5 = jnp.zeros((ACCW, ACCW), f32).at[0:300, 0:300].set(w5t.T)
    b5p = jnp.zeros((1, ACCW), f32).at[:, 0:300].set(b5[None, :])
    g5p = jnp.zeros((1, ACCW), f32).at[:, 0:300].set(g5[None, :])
    be5p = jnp.zeros((1, ACCW), f32).at[:, 0:300].set(be5[None, :])
    return _out_mlp(agg, s4[:, 0:ACCW], t4[:, 0:ACCW], w5, b5p, g5p, be5p)
